# baseline (speedup 1.0000x reference)
import functools

import jax
import jax.numpy as jnp
from jax import lax
from jax.experimental import pallas as pl
from jax.experimental.pallas import tpu as pltpu
from jax.experimental.pallas import tpu_sc as plsc

N = 10000
B = 64
E = 320000
ITERS = 20
LEAK = 0.01

NC = 2
NS = 16
Bh = B // NC
R = N // NS
K = 128
NBUF = 3
EperT = -(-E // NS)
NCH = NBUF * (-(-EperT // (NBUF * K)))
EP = NS * NCH * K
HL = Bh // 16
RC = 125


def _act(v):
    return jnp.maximum(v, 0.0) + LEAK * jnp.minimum(v, 0.0)


def _sc_body(binc, colp, rowp, wp, out, acc_sh, colv, rowv, wv,
             binv, gbuf, sbuf, gsem0, gsem1, gsem2, ssem0, ssem1, ssem2):
    c = lax.axis_index("c")
    s = lax.axis_index("s")
    gsem = (gsem0, gsem1, gsem2)
    ssem = (ssem0, ssem1, ssem2)

    pltpu.sync_copy(colp.at[s], colv)
    pltpu.sync_copy(rowp.at[s], rowv)
    pltpu.sync_copy(wp.at[s], wv)
    pltpu.sync_copy(binc.at[c, s], binv)

    for t in range(R // RC):
        def init_row(r, carry, t=t):
            for h in range(HL):
                gbuf[0, r, pl.ds(h * 16, 16)] = _act(
                    binv[t * RC + r, pl.ds(h * 16, 16)])
            return carry
        lax.fori_loop(0, RC, init_row, 0)
        pltpu.sync_copy(gbuf.at[0, pl.ds(0, RC)],
                        out.at[c, pl.ds(s * R + t * RC, RC)])
    pltpu.sync_copy(binv, acc_sh.at[pl.ds(s * R, R)])
    plsc.subcore_barrier()

    def scale(q, b):
        for g in range(K // 16):
            wvec = wv[q, pl.ds(g * 16, 16)]
            for k in range(16):
                bc = jnp.take_along_axis(
                    wvec, jnp.full((16,), k, jnp.int32), axis=0)
                r = g * 16 + k
                for h in range(HL):
                    sbuf[b, r, pl.ds(h * 16, 16)] = (
                        gbuf[b, r, pl.ds(h * 16, 16)] * bc)

    def start_gather(q, b):
        return pltpu.async_copy(out.at[c].at[colv.at[q]], gbuf.at[b],
                                gsem[b])

    def wait_gather(q, b):
        pltpu.make_async_copy(out.at[c].at[colv.at[q]], gbuf.at[b],
                              gsem[b]).wait()

    def start_scatter(q, b):
        return pltpu.async_copy(sbuf.at[b], acc_sh.at[rowv.at[q]],
                                ssem[b], add=True)

    def wait_scatter(q, b):
        pltpu.make_async_copy(sbuf.at[b], acc_sh.at[rowv.at[q]],
                              ssem[b]).wait()

    def iteration(it, carry):
        for b in range(NBUF):
            start_gather(b, b)

        def rung(i, carry2):
            for b in range(NBUF):
                q = i * NBUF + b
                wait_gather(q, b)

                @pl.when(i > 0)
                def _():
                    wait_scatter(q - NBUF, b)

                scale(q, b)

                @pl.when(q + NBUF < NCH)
                def _():
                    start_gather(q + NBUF, b)

                start_scatter(q, b)
            return carry2

        lax.fori_loop(0, NCH // NBUF, rung, 0)
        for b in range(NBUF):
            wait_scatter(NCH - NBUF + b, b)
        plsc.subcore_barrier()

        for t in range(R // RC):
            sl = pl.ds(s * R + t * RC, RC)
            pltpu.sync_copy(acc_sh.at[sl], gbuf.at[0, pl.ds(0, RC)])

            def upd_row(r, carry3):
                for h in range(HL):
                    gbuf[0, r, pl.ds(h * 16, 16)] = _act(
                        gbuf[0, r, pl.ds(h * 16, 16)])
                return carry3

            lax.fori_loop(0, RC, upd_row, 0)
            pltpu.sync_copy(gbuf.at[0, pl.ds(0, RC)], out.at[c, sl])
            pltpu.sync_copy(binv.at[pl.ds(t * RC, RC)], acc_sh.at[sl])
        plsc.subcore_barrier()
        return carry

    lax.fori_loop(0, ITERS - 1, iteration, 0)


@jax.jit
def _run(binc, colp, rowp, wp):
    f = pl.kernel(
        _sc_body,
        out_type=jax.ShapeDtypeStruct((NC, N, Bh), jnp.float32),
        mesh=plsc.VectorSubcoreMesh(core_axis_name="c", subcore_axis_name="s"),
        compiler_params=pltpu.CompilerParams(use_tc_tiling_on_sc=False),
        scratch_types=[
            pltpu.VMEM_SHARED((N, Bh), jnp.float32),
            pltpu.VMEM((NCH, K), jnp.int32),
            pltpu.VMEM((NCH, K), jnp.int32),
            pltpu.VMEM((NCH, K), jnp.float32),
            pltpu.VMEM((R, Bh), jnp.float32),
            pltpu.VMEM((NBUF, K, Bh), jnp.float32),
            pltpu.VMEM((NBUF, K, Bh), jnp.float32),
            pltpu.SemaphoreType.DMA,
            pltpu.SemaphoreType.DMA,
            pltpu.SemaphoreType.DMA,
            pltpu.SemaphoreType.DMA,
            pltpu.SemaphoreType.DMA,
            pltpu.SemaphoreType.DMA,
        ],
    )
    return f(binc, colp, rowp, wp)


def kernel(x, weights, bias, row, col):
    row = row.astype(jnp.int32)
    col = col.astype(jnp.int32)
    weights = weights.astype(jnp.float32)
    pad = EP - E
    colp = jnp.concatenate([col, jnp.zeros((pad,), jnp.int32)]).reshape(NS, NCH, K)
    rowp = jnp.concatenate([row, jnp.zeros((pad,), jnp.int32)]).reshape(NS, NCH, K)
    wp = jnp.concatenate([weights, jnp.zeros((pad,), jnp.float32)]).reshape(NS, NCH, K)
    bIn = x.T + bias
    binc = bIn.reshape(N, NC, Bh).transpose(1, 0, 2)
    binc = binc.reshape(NC, NS, R, Bh)
    out = _run(binc, colp, rowp, wp)
    return out.transpose(1, 0, 2).reshape(N, B).T


# speedup vs baseline: 2.0186x; 2.0186x over previous
import functools

import jax
import jax.numpy as jnp
from jax import lax
from jax.experimental import pallas as pl
from jax.experimental.pallas import tpu as pltpu
from jax.experimental.pallas import tpu_sc as plsc

N = 10000
B = 64
E = 320000
ITERS = 20
LEAK = 0.01

NC = 2
NS = 16
Bh = B // NC
R = N // NS
K = 128
NBUF = 2
EperT = -(-E // NS)
NCH = NBUF * (-(-EperT // (NBUF * K)))
EP = NS * NCH * K
HL = Bh // 16
RC = 125
NT = R // RC


def _act(v):
    return jnp.maximum(v, 0.0) + LEAK * jnp.minimum(v, 0.0)


def _sc_body(binc, colp, rowp, wp, out, xh_sh, acc_sh, colv, rowv, wv,
             gbuf, sbuf, bbuf, gsem0, gsem1, ssem0, ssem1, bsem):
    c = lax.axis_index("c")
    s = lax.axis_index("s")
    gsem = (gsem0, gsem1)
    ssem = (ssem0, ssem1)

    pltpu.sync_copy(colp.at[s], colv)
    pltpu.sync_copy(rowp.at[s], rowv)
    pltpu.sync_copy(wp.at[s], wv)

    for t in range(NT):
        sl = pl.ds(s * R + t * RC, RC)
        pltpu.sync_copy(binc.at[c, s, pl.ds(t * RC, RC)], bbuf.at[0])

        def init_row(r, carry):
            for h in range(HL):
                gbuf[0, r, pl.ds(h * 16, 16)] = _act(
                    bbuf[0, r, pl.ds(h * 16, 16)])
            return carry

        lax.fori_loop(0, RC, init_row, 0)
        pltpu.sync_copy(gbuf.at[0, pl.ds(0, RC)], xh_sh.at[sl])
        pltpu.sync_copy(bbuf.at[0], acc_sh.at[sl])
    plsc.subcore_barrier()

    def scale(q, b):
        for g in range(K // 16):
            wvec = wv[q, pl.ds(g * 16, 16)]
            for k in range(16):
                bc = jnp.take_along_axis(
                    wvec, jnp.full((16,), k, jnp.int32), axis=0)
                r = g * 16 + k
                for h in range(HL):
                    sbuf[b, r, pl.ds(h * 16, 16)] = (
                        gbuf[b, r, pl.ds(h * 16, 16)] * bc)

    def start_gather(q, b):
        return pltpu.async_copy(xh_sh.at[colv.at[q]], gbuf.at[b], gsem[b])

    def wait_gather(q, b):
        pltpu.make_async_copy(xh_sh.at[colv.at[q]], gbuf.at[b],
                              gsem[b]).wait()

    def start_scatter(q, b):
        return pltpu.async_copy(sbuf.at[b], acc_sh.at[rowv.at[q]],
                                ssem[b], add=True)

    def wait_scatter(q, b):
        pltpu.make_async_copy(sbuf.at[b], acc_sh.at[rowv.at[q]],
                              ssem[b]).wait()

    def start_bin(t, tb):
        return pltpu.async_copy(binc.at[c, s, pl.ds(t * RC, RC)],
                                bbuf.at[tb], bsem)

    def wait_bin(t, tb):
        pltpu.make_async_copy(binc.at[c, s, pl.ds(t * RC, RC)],
                              bbuf.at[tb], bsem).wait()

    def iteration(it, carry):
        for b in range(NBUF):
            start_gather(b, b)

        def rung(i, carry2):
            for b in range(NBUF):
                q = i * NBUF + b
                wait_gather(q, b)

                @pl.when(i > 0)
                def _():
                    wait_scatter(q - NBUF, b)

                scale(q, b)

                @pl.when(q + NBUF < NCH)
                def _():
                    start_gather(q + NBUF, b)

                start_scatter(q, b)
            return carry2

        lax.fori_loop(0, NCH // NBUF, rung, 0)
        start_bin(0, 0)
        for b in range(NBUF):
            wait_scatter(NCH - NBUF + b, b)
        plsc.subcore_barrier()

        for t in range(NT):
            tb = t % 2
            sl = pl.ds(s * R + t * RC, RC)
            wait_bin(t, tb)
            if t + 1 < NT:
                start_bin(t + 1, 1 - tb)
            pltpu.sync_copy(acc_sh.at[sl], gbuf.at[0, pl.ds(0, RC)])

            def upd_row(r, carry3):
                for h in range(HL):
                    gbuf[0, r, pl.ds(h * 16, 16)] = _act(
                        gbuf[0, r, pl.ds(h * 16, 16)])
                return carry3

            lax.fori_loop(0, RC, upd_row, 0)
            pltpu.sync_copy(gbuf.at[0, pl.ds(0, RC)], xh_sh.at[sl])
            pltpu.sync_copy(bbuf.at[tb], acc_sh.at[sl])
        plsc.subcore_barrier()
        return carry

    lax.fori_loop(0, ITERS - 1, iteration, 0)
    pltpu.sync_copy(xh_sh.at[pl.ds(s * R, R)], out.at[c, pl.ds(s * R, R)])


@jax.jit
def _run(binc, colp, rowp, wp):
    f = pl.kernel(
        _sc_body,
        out_type=jax.ShapeDtypeStruct((NC, N, Bh), jnp.float32),
        mesh=plsc.VectorSubcoreMesh(core_axis_name="c", subcore_axis_name="s"),
        compiler_params=pltpu.CompilerParams(use_tc_tiling_on_sc=False),
        scratch_types=[
            pltpu.VMEM_SHARED((N, Bh), jnp.float32),
            pltpu.VMEM_SHARED((N, Bh), jnp.float32),
            pltpu.VMEM((NCH, K), jnp.int32),
            pltpu.VMEM((NCH, K), jnp.int32),
            pltpu.VMEM((NCH, K), jnp.float32),
            pltpu.VMEM((NBUF, K, Bh), jnp.float32),
            pltpu.VMEM((NBUF, K, Bh), jnp.float32),
            pltpu.VMEM((2, RC, Bh), jnp.float32),
            pltpu.SemaphoreType.DMA,
            pltpu.SemaphoreType.DMA,
            pltpu.SemaphoreType.DMA,
            pltpu.SemaphoreType.DMA,
            pltpu.SemaphoreType.DMA,
        ],
    )
    return f(binc, colp, rowp, wp)


def kernel(x, weights, bias, row, col):
    row = row.astype(jnp.int32)
    col = col.astype(jnp.int32)
    weights = weights.astype(jnp.float32)
    pad = EP - E
    colp = jnp.concatenate([col, jnp.zeros((pad,), jnp.int32)]).reshape(NS, NCH, K)
    rowp = jnp.concatenate([row, jnp.zeros((pad,), jnp.int32)]).reshape(NS, NCH, K)
    wp = jnp.concatenate([weights, jnp.zeros((pad,), jnp.float32)]).reshape(NS, NCH, K)
    bIn = x.T + bias
    binc = bIn.reshape(N, NC, Bh).transpose(1, 0, 2)
    binc = binc.reshape(NC, NS, R, Bh)
    out = _run(binc, colp, rowp, wp)
    return out.transpose(1, 0, 2).reshape(N, B).T


# speedup vs baseline: 2.4113x; 1.1945x over previous
import functools

import jax
import jax.numpy as jnp
from jax import lax
from jax.experimental import pallas as pl
from jax.experimental.pallas import tpu as pltpu
from jax.experimental.pallas import tpu_sc as plsc

N = 10000
B = 64
E = 320000
ITERS = 20
LEAK = 0.01

NC = 2
NS = 16
Bh = B // NC
R = N // NS
K = 128
NBUF = 2
EperT = -(-E // NS)
NCH = NBUF * (-(-EperT // (NBUF * K)))
EP = NS * NCH * K
HL = Bh // 16
RC = 125
NT = R // RC


def _act(v):
    return jnp.maximum(v, 0.0) + LEAK * jnp.minimum(v, 0.0)


def _pack(a, b):
    return plsc.pack(a, b, format=plsc.PackFormat.INTERLEAVED)


def _unpack(ab):
    return plsc.unpack(ab, format=plsc.PackFormat.INTERLEAVED)


def _sc_body(binc, colp, rowp, wp, out, xh_sh, acc_sh, colv, rowv, wv,
             gbuf, sbuf, bbuf, ubuf, xbuf, gsem0, gsem1, ssem0, ssem1, bsem):
    c = lax.axis_index("c")
    s = lax.axis_index("s")
    gsem = (gsem0, gsem1)
    ssem = (ssem0, ssem1)

    pltpu.sync_copy(colp.at[s], colv)
    pltpu.sync_copy(rowp.at[s], rowv)
    pltpu.sync_copy(wp.at[s], wv)

    for t in range(NT):
        sl = pl.ds(s * R + t * RC, RC)
        pltpu.sync_copy(binc.at[c, s, pl.ds(t * RC, RC)], bbuf.at[0])

        def init_row(r, carry):
            v0 = _act(bbuf[0, r, pl.ds(0, 16)])
            v1 = _act(bbuf[0, r, pl.ds(16, 16)])
            xbuf[r, pl.ds(0, 32)] = _pack(v0, v1)
            return carry

        lax.fori_loop(0, RC, init_row, 0)
        pltpu.sync_copy(xbuf, xh_sh.at[sl])
        pltpu.sync_copy(bbuf.at[0], acc_sh.at[sl])
    plsc.subcore_barrier()

    def scale(q, b):
        for g in range(K // 16):
            wvec = wv[q, pl.ds(g * 16, 16)]
            for k in range(16):
                bc = jnp.take_along_axis(
                    wvec, jnp.full((16,), k, jnp.int32), axis=0)
                r = g * 16 + k
                v0, v1 = _unpack(gbuf[b, r, pl.ds(0, 32)])
                sbuf[b, r, pl.ds(0, 16)] = v0 * bc
                sbuf[b, r, pl.ds(16, 16)] = v1 * bc

    def start_gather(q, b):
        return pltpu.async_copy(xh_sh.at[colv.at[q]], gbuf.at[b], gsem[b])

    def wait_gather(q, b):
        pltpu.make_async_copy(xh_sh.at[colv.at[q]], gbuf.at[b],
                              gsem[b]).wait()

    def start_scatter(q, b):
        return pltpu.async_copy(sbuf.at[b], acc_sh.at[rowv.at[q]],
                                ssem[b], add=True)

    def wait_scatter(q, b):
        pltpu.make_async_copy(sbuf.at[b], acc_sh.at[rowv.at[q]],
                              ssem[b]).wait()

    def start_bin(t, tb):
        return pltpu.async_copy(binc.at[c, s, pl.ds(t * RC, RC)],
                                bbuf.at[tb], bsem)

    def wait_bin(t, tb):
        pltpu.make_async_copy(binc.at[c, s, pl.ds(t * RC, RC)],
                              bbuf.at[tb], bsem).wait()

    def iteration(it, carry):
        for b in range(NBUF):
            start_gather(b, b)

        def rung(i, carry2):
            for b in range(NBUF):
                q = i * NBUF + b
                wait_gather(q, b)

                @pl.when(i > 0)
                def _():
                    wait_scatter(q - NBUF, b)

                scale(q, b)

                @pl.when(q + NBUF < NCH)
                def _():
                    start_gather(q + NBUF, b)

                start_scatter(q, b)
            return carry2

        lax.fori_loop(0, NCH // NBUF, rung, 0)
        start_bin(0, 0)
        for b in range(NBUF):
            wait_scatter(NCH - NBUF + b, b)
        plsc.subcore_barrier()

        for t in range(NT):
            tb = t % 2
            sl = pl.ds(s * R + t * RC, RC)
            wait_bin(t, tb)
            if t + 1 < NT:
                start_bin(t + 1, 1 - tb)
            pltpu.sync_copy(acc_sh.at[sl], ubuf)

            def upd_row(r, carry3):
                v0 = _act(ubuf[r, pl.ds(0, 16)])
                v1 = _act(ubuf[r, pl.ds(16, 16)])
                xbuf[r, pl.ds(0, 32)] = _pack(v0, v1)
                ubuf[r, pl.ds(0, 16)] = v0
                ubuf[r, pl.ds(16, 16)] = v1
                return carry3

            lax.fori_loop(0, RC, upd_row, 0)
            pltpu.sync_copy(xbuf, xh_sh.at[sl])

            @pl.when(it == ITERS - 2)
            def _():
                pltpu.sync_copy(ubuf, out.at[c, pl.ds(s * R + t * RC, RC)])

            pltpu.sync_copy(bbuf.at[tb], acc_sh.at[sl])
        plsc.subcore_barrier()
        return carry

    lax.fori_loop(0, ITERS - 1, iteration, 0)


@jax.jit
def _run(binc, colp, rowp, wp):
    f = pl.kernel(
        _sc_body,
        out_type=jax.ShapeDtypeStruct((NC, N, Bh), jnp.float32),
        mesh=plsc.VectorSubcoreMesh(core_axis_name="c", subcore_axis_name="s"),
        compiler_params=pltpu.CompilerParams(use_tc_tiling_on_sc=False,
                                             needs_layout_passes=False),
        scratch_types=[
            pltpu.VMEM_SHARED((N, Bh), jnp.bfloat16),
            pltpu.VMEM_SHARED((N, Bh), jnp.float32),
            pltpu.VMEM((NCH, K), jnp.int32),
            pltpu.VMEM((NCH, K), jnp.int32),
            pltpu.VMEM((NCH, K), jnp.float32),
            pltpu.VMEM((NBUF, K, Bh), jnp.bfloat16),
            pltpu.VMEM((NBUF, K, Bh), jnp.float32),
            pltpu.VMEM((2, RC, Bh), jnp.float32),
            pltpu.VMEM((RC, Bh), jnp.float32),
            pltpu.VMEM((RC, Bh), jnp.bfloat16),
            pltpu.SemaphoreType.DMA,
            pltpu.SemaphoreType.DMA,
            pltpu.SemaphoreType.DMA,
            pltpu.SemaphoreType.DMA,
            pltpu.SemaphoreType.DMA,
        ],
    )
    return f(binc, colp, rowp, wp)


def kernel(x, weights, bias, row, col):
    row = row.astype(jnp.int32)
    col = col.astype(jnp.int32)
    weights = weights.astype(jnp.float32)
    pad = EP - E
    colp = jnp.concatenate([col, jnp.zeros((pad,), jnp.int32)]).reshape(NS, NCH, K)
    rowp = jnp.concatenate([row, jnp.zeros((pad,), jnp.int32)]).reshape(NS, NCH, K)
    wp = jnp.concatenate([weights, jnp.zeros((pad,), jnp.float32)]).reshape(NS, NCH, K)
    bIn = x.T + bias
    binc = bIn.reshape(N, NC, Bh).transpose(1, 0, 2)
    binc = binc.reshape(NC, NS, R, Bh)
    out = _run(binc, colp, rowp, wp)
    return out.transpose(1, 0, 2).reshape(N, B).T
